# Initial kernel scaffold; baseline (speedup 1.0000x reference)
import jax
import jax.numpy as jnp
from jax.experimental import pallas as pl


def kernel(token, targets, table):
    raise NotImplementedError("write your pallas kernel here")


# speedup vs baseline: 2.4523x; 2.4523x over previous
import functools

import jax
import jax.numpy as jnp
from jax import lax
from jax.experimental import pallas as pl
from jax.experimental.pallas import tpu as pltpu
from jax.experimental.pallas import tpu_sc as plsc

VOCAB = 4096
NTOK = 16384

NC = 2
NS = 16
NW = NC * NS
BPW = NTOK // NW
K = 8
NCHUNK = BPW // K


def _lse_table_tc(table):
    bv = 256

    def body(t_ref, o_ref):
        x = t_ref[...]
        m = jnp.max(x, axis=-1)
        s = jnp.sum(jnp.exp(x - m[:, None]), axis=-1)
        o_ref[...] = m + jnp.log(s)

    return pl.pallas_call(
        body,
        grid=(VOCAB // bv,),
        in_specs=[pl.BlockSpec((bv, VOCAB), lambda i: (i, 0))],
        out_specs=pl.BlockSpec((bv,), lambda i: (i,)),
        out_shape=jax.ShapeDtypeStruct((VOCAB,), jnp.float32),
    )(table)


def _sc_gather(tok, tgt, table, lse_t):
    mesh = plsc.VectorSubcoreMesh(
        core_axis_name="c", subcore_axis_name="s",
        num_cores=NC, num_subcores=NS)

    @functools.partial(
        pl.kernel,
        out_type=[
            jax.ShapeDtypeStruct((NTOK, VOCAB), jnp.float32),
            jax.ShapeDtypeStruct((NW * 16,), jnp.float32),
            jax.ShapeDtypeStruct((NW * 16,), jnp.float32),
        ],
        mesh=mesh,
        compiler_params=pltpu.CompilerParams(needs_layout_passes=False),
        scratch_types=[
            pltpu.VMEM((BPW,), jnp.int32),
            pltpu.VMEM((BPW,), jnp.int32),
            pltpu.VMEM((VOCAB,), jnp.float32),
            pltpu.VMEM((K, VOCAB), jnp.float32),
            pltpu.VMEM((K, VOCAB), jnp.float32),
            pltpu.VMEM((16,), jnp.float32),
            pltpu.VMEM((16,), jnp.float32),
            pltpu.SemaphoreType.DMA,
            pltpu.SemaphoreType.DMA,
            pltpu.SemaphoreType.DMA,
            pltpu.SemaphoreType.DMA,
        ],
    )
    def k(tok_hbm, tgt_hbm, tbl_hbm, lse_hbm, out_hbm, lsep_hbm, tlp_hbm,
          idx_v, tgt_v, lse_v, buf0, buf1, st0, st1,
          gsem0, gsem1, osem0, osem1):
        wid = lax.axis_index("s") * NC + lax.axis_index("c")
        base = pl.multiple_of(wid * BPW, BPW)

        pltpu.sync_copy(tok_hbm.at[pl.ds(base, BPW)], idx_v)
        pltpu.sync_copy(tgt_hbm.at[pl.ds(base, BPW)], tgt_v)
        pltpu.sync_copy(lse_hbm, lse_v)

        bufs = (buf0, buf1)
        gsems = (gsem0, gsem1)
        osems = (osem0, osem1)
        lane = lax.iota(jnp.int32, 16)
        rowsel = lane & (K - 1)

        def g_desc(g, b):
            off = pl.multiple_of(g * K, 8)
            return pltpu.make_async_copy(
                tbl_hbm.at[idx_v.at[pl.ds(off, K)]], bufs[b], gsems[b])

        def o_desc(g, b):
            roff = pl.multiple_of(base + g * K, 8)
            return pltpu.make_async_copy(
                bufs[b], out_hbm.at[pl.ds(roff, K)], osems[b])

        def extract(g, b, half, acc):
            toff = pl.multiple_of((g // 2) * 16, 8)
            t16 = tgt_v[pl.ds(toff, 16)]
            v = plsc.load_gather(bufs[b], [rowsel, t16])
            if half == 0:
                sel = lane < K
            else:
                sel = lane >= K
            return acc + jnp.where(sel, v, jnp.float32(0.0))

        g_desc(0, 0).start()
        g_desc(1, 1).start()

        def lse_loop(j, acc):
            off = pl.multiple_of(j * 16, 8)
            t16 = idx_v[pl.ds(off, 16)]
            return acc + plsc.load_gather(lse_v, [t16])

        lse_acc = lax.fori_loop(0, BPW // 16, lse_loop,
                                jnp.zeros((16,), jnp.float32))

        g_desc(0, 0).wait()
        tl_acc = extract(0, 0, 0, jnp.zeros((16,), jnp.float32))
        o_desc(0, 0).start()

        def outer(j, acc):
            for c in (1, 2):
                g = 2 * j + c
                b = c & 1
                nb = 1 - b
                o_desc(g - 1, nb).wait()
                g_desc(g + 1, nb).start()
                g_desc(g, b).wait()
                acc = extract(g, b, b, acc)
                o_desc(g, b).start()
            return acc

        tl_acc = lax.fori_loop(0, (NCHUNK - 2) // 2, outer, tl_acc)

        g_last = NCHUNK - 1
        o_desc(g_last - 1, 0).wait()
        g_desc(g_last, 1).wait()
        tl_acc = extract(g_last, 1, 1, tl_acc)
        o_desc(g_last, 1).start()
        o_desc(g_last, 1).wait()

        st0[...] = lse_acc
        st1[...] = tl_acc
        poff = pl.multiple_of(wid * 16, 16)
        pltpu.sync_copy(st0, lsep_hbm.at[pl.ds(poff, 16)])
        pltpu.sync_copy(st1, tlp_hbm.at[pl.ds(poff, 16)])

    return k(tok, tgt, table, lse_t)


def _finish_tc(lse_parts, tl_parts):
    def body(a_ref, b_ref, o_ref):
        o_ref[0, 0] = (jnp.sum(a_ref[...]) - jnp.sum(b_ref[...])) / NTOK

    out = pl.pallas_call(
        body,
        out_specs=pl.BlockSpec(memory_space=pltpu.SMEM),
        out_shape=jax.ShapeDtypeStruct((1, 1), jnp.float32),
    )(lse_parts.reshape(4, 128), tl_parts.reshape(4, 128))
    return out[0, 0]


def kernel(token, targets, table):
    n, c = token.shape
    tok = token.reshape(-1)
    tgt = targets.reshape(-1)
    lse_t = _lse_table_tc(table)
    logits_flat, lse_p, tl_p = _sc_gather(tok, tgt, table, lse_t)
    loss = _finish_tc(lse_p, tl_p)
    return logits_flat.reshape(n, c, VOCAB), loss
